# Initial kernel scaffold; baseline (speedup 1.0000x reference)
import jax
import jax.numpy as jnp
from jax.experimental import pallas as pl


def kernel(x, edge_index, Ws1, Ws2, Wm, Ws):
    raise NotImplementedError("write your pallas kernel here")


# speedup vs baseline: 11.4236x; 11.4236x over previous
import functools

import jax
import jax.numpy as jnp
from jax import lax
from jax.experimental import pallas as pl
from jax.experimental.pallas import tpu as pltpu
from jax.experimental.pallas import tpu_sc as plsc

N = 10000
E = 320000
D = 128
R = 4
H = 64
LD = 32

NC = 2
NS = 16
EC = 128
ROWS2D = E // EC
CPT = 160
CPT2 = 80
ROWSPAD = NS * CPT
ZR = 200
NZC = N // ZR
NP = 10240


def _sc_mesh():
    return plsc.VectorSubcoreMesh(
        core_axis_name="c", subcore_axis_name="s", num_cores=NC, num_subcores=NS
    )


def _zero_acc(s, zrows, acc):
    for j in range(4):
        idx = s + NS * j

        @pl.when(idx < NZC)
        def _():
            pltpu.sync_copy(zrows, acc.at[pl.ds(idx * ZR, ZR)])


def _write_out(s, acc, outh):
    for j in range(4):
        idx = s + NS * j

        @pl.when(idx < NZC)
        def _():
            pltpu.sync_copy(acc.at[pl.ds(idx * ZR, ZR)], outh.at[pl.ds(idx * ZR, ZR)])


G = 32


def _agg_loop(nck, tilebase, xh, src2d, dst2d, srcb, dstb, rows, acc,
              isem, gsem, ssem):
    def idx_start(g, p):
        base = tilebase + g * G
        pltpu.async_copy(src2d.at[pl.ds(base, G)], srcb.at[p], isem.at[p])
        pltpu.async_copy(dst2d.at[pl.ds(base, G)], dstb.at[p], isem.at[p])

    def idx_wait(g, p):
        base = tilebase + g * G
        pltpu.make_async_copy(src2d.at[pl.ds(base, G)], srcb.at[p], isem.at[p]).wait()
        pltpu.make_async_copy(dst2d.at[pl.ds(base, G)], dstb.at[p], isem.at[p]).wait()

    def g_start(k, p, j, b):
        pltpu.async_copy(xh.at[srcb.at[p, j]], rows.at[b], gsem.at[b])

    def g_wait(k, p, j, b):
        pltpu.make_async_copy(xh.at[srcb.at[p, j]], rows.at[b], gsem.at[b]).wait()

    idx_start(0, 0)

    def body(k, _):
        j = jnp.bitwise_and(k, G - 1)
        g = lax.shift_right_logical(k, 5)
        p = jnp.bitwise_and(g, 1)
        b = jnp.bitwise_and(k, 1)

        @pl.when(j == 0)
        def _():
            idx_wait(g, p)
            pl.when((g + 1) * G < nck)(lambda: idx_start(g + 1, 1 - p))
            g_start(k, p, 0, b)
            pl.when(k + 1 < nck)(lambda: g_start(k + 1, p, 1, 1 - b))

        g_wait(k, p, j, b)
        pltpu.async_copy(rows.at[b], acc.at[dstb.at[p, j]], ssem.at[b], add=True)
        pltpu.make_async_copy(rows.at[b], acc.at[dstb.at[p, j]], ssem.at[b]).wait()
        pl.when(jnp.logical_and(j < G - 2, k + 2 < nck))(
            lambda: g_start(k + 2, p, j + 2, b)
        )
        return _

    lax.fori_loop(0, nck, body, None)


_AGG_SCRATCH = [
    pltpu.VMEM((2, G, EC), jnp.int32),
    pltpu.VMEM((2, G, EC), jnp.int32),
    pltpu.VMEM((2, EC, D), jnp.float32),
    pltpu.VMEM_SHARED((N, D), jnp.float32),
    pltpu.SemaphoreType.DMA((2,)),
    pltpu.SemaphoreType.DMA((2,)),
    pltpu.SemaphoreType.DMA((2,)),
]


def _make_agg_esplit():
    @functools.partial(
        pl.kernel,
        out_type=(
            jax.ShapeDtypeStruct((N, D), jnp.float32),
            jax.ShapeDtypeStruct((N, D), jnp.float32),
        ),
        mesh=_sc_mesh(),
        scratch_types=_AGG_SCRATCH,
    )
    def agg(x, src2d, dst2d, zrows, out0, out1, srcb, dstb, rows, acc, isem, gsem, ssem):
        c = lax.axis_index("c")
        s = lax.axis_index("s")
        w = c * NS + s

        _zero_acc(s, zrows, acc)
        nck = jnp.clip(ROWS2D - w * CPT2, 0, CPT2)

        plsc.subcore_barrier()
        _agg_loop(nck, w * CPT2, x, src2d, dst2d, srcb, dstb, rows, acc,
                  isem, gsem, ssem)
        plsc.subcore_barrier()

        pl.when(c == 0)(lambda: _write_out(s, acc, out0))
        pl.when(c == 1)(lambda: _write_out(s, acc, out1))

    return agg


def _make_agg_fsplit():
    @functools.partial(
        pl.kernel,
        out_type=(
            jax.ShapeDtypeStruct((N, D), jnp.float32),
            jax.ShapeDtypeStruct((N, D), jnp.float32),
        ),
        mesh=_sc_mesh(),
        scratch_types=_AGG_SCRATCH,
    )
    def agg(x0, x1, src2d, dst2d, zrows, out0, out1, srcb, dstb, rows, acc,
            isem, gsem, ssem):
        c = lax.axis_index("c")
        s = lax.axis_index("s")

        _zero_acc(s, zrows, acc)
        nck = jnp.clip(ROWS2D - s * CPT, 0, CPT)

        plsc.subcore_barrier()

        def run(xh, outh):
            _agg_loop(nck, s * CPT, xh, src2d, dst2d, srcb, dstb, rows, acc,
                      isem, gsem, ssem)
            plsc.subcore_barrier()
            _write_out(s, acc, outh)

        pl.when(c == 0)(lambda: run(x0, out0))
        pl.when(c == 1)(lambda: run(x1, out1))

    return agg


def _make_deg():
    @functools.partial(
        pl.kernel,
        out_type=(
            jax.ShapeDtypeStruct((NS * NP,), jnp.float32),
            jax.ShapeDtypeStruct((NS * NP,), jnp.float32),
        ),
        mesh=_sc_mesh(),
        scratch_types=[
            pltpu.VMEM((CPT, EC), jnp.int32),
            pltpu.VMEM((NP,), jnp.float32),
        ],
        compiler_params=pltpu.CompilerParams(needs_layout_passes=False),
    )
    def deg(src2d, dst2d, zflat, out_s, out_d, idxb, counts):
        c = lax.axis_index("c")
        s = lax.axis_index("s")

        pltpu.sync_copy(zflat, counts)
        pl.when(c == 0)(lambda: pltpu.sync_copy(src2d.at[pl.ds(s * CPT, CPT)], idxb))
        pl.when(c == 1)(lambda: pltpu.sync_copy(dst2d.at[pl.ds(s * CPT, CPT)], idxb))
        nck = jnp.clip(ROWS2D - s * CPT, 0, CPT)

        ones16 = jnp.full((16,), 1.0, jnp.float32)

        def body(k, _):
            for j in range(EC // 16):
                v = idxb[k, pl.ds(j * 16, 16)]
                plsc.addupdate_scatter(counts, [v], ones16)
            return _

        lax.fori_loop(0, nck, body, None)

        sl = pl.ds(s * NP, NP)
        pl.when(c == 0)(lambda: pltpu.sync_copy(counts, out_s.at[sl]))
        pl.when(c == 1)(lambda: pltpu.sync_copy(counts, out_d.at[sl]))

    return deg


def _dot(a, b):
    return jnp.dot(a, b, preferred_element_type=jnp.float32,
                   precision=lax.Precision.HIGHEST)


def _tc_call(body, out_shapes):
    return pl.pallas_call(
        body,
        out_shape=tuple(jax.ShapeDtypeStruct(s, jnp.float32) for s in out_shapes),
    )


BR = 1000


def _rb_spec(shape):
    if shape[0] == N:
        nd = len(shape)
        return pl.BlockSpec((BR,) + shape[1:], lambda i: (i,) + (0,) * (nd - 1))
    return pl.BlockSpec(shape, lambda i: (0,) * len(shape))


def _tc_rowblock(body, in_shapes, out_shapes):
    return pl.pallas_call(
        body,
        grid=(N // BR,),
        in_specs=[_rb_spec(s) for s in in_shapes],
        out_specs=tuple(_rb_spec(s) for s in out_shapes),
        out_shape=tuple(jax.ShapeDtypeStruct(s, jnp.float32) for s in out_shapes),
    )


def _degsum_body(csr, cdr, gs_r, gd_r):
    gs_r[...] = jnp.sum(csr[...], axis=0)
    gd_r[...] = jnp.sum(cdr[...], axis=0)


def _prep_body(dsr, ddr, xr, inv_s_r, inv_d_r, xs_r):
    ds = dsr[...]
    dd = ddr[...]
    inv_s = jnp.where(ds > 0, lax.rsqrt(jnp.maximum(ds, 1.0)), 0.0)
    inv_d = jnp.where(dd > 0, lax.rsqrt(jnp.maximum(dd, 1.0)), 0.0)
    inv_s_r[...] = inv_s
    inv_d_r[...] = inv_d
    xs_r[...] = xr[...] * inv_s


def _mix1_body(p0r, p1r, invdr, invsr, wr, g0r, g1r):
    y = (p0r[...] + p1r[...]) * invdr[...]
    g = jnp.tanh(_dot(y, wr[...])) * invsr[...]
    g0r[...] = g[:, :D]
    g1r[...] = g[:, D:]


def _mix2_body(y0r, y1r, invdr, invsr, w2r, wcr, qr):
    y = jnp.concatenate([y0r[...], y1r[...]], axis=1) * invdr[...]
    z = jnp.tanh(_dot(y, w2r[...])) * invsr[...]
    qr[...] = _dot(z, wcr[...])


def _fin_body(y0r, y1r, invdr, zmr, zvr):
    yw = (y0r[...] + y1r[...]) * invdr[...]
    p = yw[:, :LD]
    nrm = jnp.sqrt(jnp.sum(p * p, axis=1, keepdims=True))
    zmr[...] = p / (1e-4 + nrm)
    v = yw[:, LD:LD + 1]
    zvr[...] = jnp.log1p(jnp.exp(-jnp.abs(v))) + jnp.maximum(v, 0.0) + 1.0


@jax.jit
def kernel(x, edge_index, Ws1, Ws2, Wm, Ws):
    ei = edge_index.astype(jnp.int32)
    pad = jnp.zeros((2, ROWSPAD * EC - E), jnp.int32)
    ei = jnp.concatenate([ei, pad], axis=1)
    src2d = ei[0].reshape(ROWSPAD, EC)
    dst2d = ei[1].reshape(ROWSPAD, EC)

    w1cat = jnp.concatenate(Ws1, axis=1)
    w2bd = jnp.zeros((R * H, R * H), jnp.float32)
    for r in range(R):
        w2bd = w2bd.at[r * H:(r + 1) * H, r * H:(r + 1) * H].set(Ws2[r])
    wcat = jnp.zeros((R * H, D), jnp.float32)
    wcat = wcat.at[:, :LD].set(Wm).at[:, LD].set(Ws[:, 0])

    zflat = jnp.zeros((NP,), jnp.float32)
    z128 = jnp.zeros((ZR, D), jnp.float32)

    ones_e = jnp.ones((E,), jnp.float32)
    deg_s = jax.ops.segment_sum(ones_e, ei[0][:E], num_segments=N).reshape(N, 1)
    deg_d = jax.ops.segment_sum(ones_e, ei[1][:E], num_segments=N).reshape(N, 1)

    inv_s, inv_d, xs = _tc_rowblock(
        _prep_body, [(N, 1), (N, 1), (N, D)], [(N, 1), (N, 1), (N, D)]
    )(deg_s, deg_d, x)

    p0, p1 = _make_agg_esplit()(xs, src2d, dst2d, z128)

    g0, g1 = _tc_rowblock(
        _mix1_body,
        [(N, D), (N, D), (N, 1), (N, 1), (D, R * H)],
        [(N, D), (N, D)],
    )(p0, p1, inv_d, inv_s, w1cat)

    y20, y21 = _make_agg_fsplit()(g0, g1, src2d, dst2d, z128)

    qp = _tc_rowblock(
        _mix2_body,
        [(N, D), (N, D), (N, 1), (N, 1), (R * H, R * H), (R * H, D)],
        [(N, D)],
    )(y20, y21, inv_d, inv_s, w2bd, wcat)[0]

    y3a, y3b = _make_agg_esplit()(qp, src2d, dst2d, z128)

    z_mean, z_var = _tc_rowblock(
        _fin_body,
        [(N, D), (N, D), (N, 1)],
        [(N, LD), (N, 1)],
    )(y3a, y3b, inv_d)

    return z_mean, z_mean, z_var


# speedup vs baseline: 23.0963x; 2.0218x over previous
import functools

import jax
import jax.numpy as jnp
from jax import lax
from jax.experimental import pallas as pl
from jax.experimental.pallas import tpu as pltpu
from jax.experimental.pallas import tpu_sc as plsc

N = 10000
E = 320000
D = 128
R = 4
H = 64
LD = 32

NC = 2
NS = 16
EC = 128
ROWS2D = E // EC
CPT = 160
CPT2 = 80
ROWSPAD = NS * CPT
ZR = 200
NZC = N // ZR
NP = 10240


def _sc_mesh():
    return plsc.VectorSubcoreMesh(
        core_axis_name="c", subcore_axis_name="s", num_cores=NC, num_subcores=NS
    )


def _zero_acc(s, zrows, acc):
    for j in range(4):
        idx = s + NS * j

        @pl.when(idx < NZC)
        def _():
            pltpu.sync_copy(zrows, acc.at[pl.ds(idx * ZR, ZR)])


def _write_out(s, acc, outh):
    for j in range(4):
        idx = s + NS * j

        @pl.when(idx < NZC)
        def _():
            pltpu.sync_copy(acc.at[pl.ds(idx * ZR, ZR)], outh.at[pl.ds(idx * ZR, ZR)])


G = 32


def _agg_loop(nck, tilebase, xh, src2d, dst2d, srcb, dstb, rows, acc,
              isem, gsem, ssem):
    def idx_start(g, p):
        base = tilebase + g * G
        pltpu.async_copy(src2d.at[pl.ds(base, G)], srcb.at[p], isem.at[p])
        pltpu.async_copy(dst2d.at[pl.ds(base, G)], dstb.at[p], isem.at[p])

    def idx_wait(g, p):
        base = tilebase + g * G
        pltpu.make_async_copy(src2d.at[pl.ds(base, G)], srcb.at[p], isem.at[p]).wait()
        pltpu.make_async_copy(dst2d.at[pl.ds(base, G)], dstb.at[p], isem.at[p]).wait()

    def g_start(k, p, j, b):
        pltpu.async_copy(xh.at[srcb.at[p, j]], rows.at[b], gsem.at[b])

    def g_wait(k, p, j, b):
        pltpu.make_async_copy(xh.at[srcb.at[p, j]], rows.at[b], gsem.at[b]).wait()

    idx_start(0, 0)

    def body(k, _):
        j = jnp.bitwise_and(k, G - 1)
        g = lax.shift_right_logical(k, 5)
        p = jnp.bitwise_and(g, 1)
        b = jnp.bitwise_and(k, 1)

        @pl.when(j == 0)
        def _():
            idx_wait(g, p)
            pl.when((g + 1) * G < nck)(lambda: idx_start(g + 1, 1 - p))
            g_start(k, p, 0, b)
            pl.when(k + 1 < nck)(lambda: g_start(k + 1, p, 1, 1 - b))

        g_wait(k, p, j, b)
        pltpu.async_copy(rows.at[b], acc.at[dstb.at[p, j]], ssem.at[b], add=True)
        pltpu.make_async_copy(rows.at[b], acc.at[dstb.at[p, j]], ssem.at[b]).wait()
        pl.when(jnp.logical_and(j < G - 2, k + 2 < nck))(
            lambda: g_start(k + 2, p, j + 2, b)
        )
        return _

    lax.fori_loop(0, nck, body, None)


_AGG_SCRATCH = [
    pltpu.VMEM((2, G, EC), jnp.int32),
    pltpu.VMEM((2, G, EC), jnp.int32),
    pltpu.VMEM((2, EC, D), jnp.float32),
    pltpu.VMEM_SHARED((N, D), jnp.float32),
    pltpu.SemaphoreType.DMA((2,)),
    pltpu.SemaphoreType.DMA((2,)),
    pltpu.SemaphoreType.DMA((2,)),
]


def _make_agg_esplit():
    @functools.partial(
        pl.kernel,
        out_type=(
            jax.ShapeDtypeStruct((N, D), jnp.float32),
            jax.ShapeDtypeStruct((N, D), jnp.float32),
        ),
        mesh=_sc_mesh(),
        scratch_types=_AGG_SCRATCH,
    )
    def agg(x, src2d, dst2d, zrows, out0, out1, srcb, dstb, rows, acc, isem, gsem, ssem):
        c = lax.axis_index("c")
        s = lax.axis_index("s")
        w = c * NS + s

        _zero_acc(s, zrows, acc)
        nck = jnp.clip(ROWS2D - w * CPT2, 0, CPT2)

        plsc.subcore_barrier()
        _agg_loop(nck, w * CPT2, x, src2d, dst2d, srcb, dstb, rows, acc,
                  isem, gsem, ssem)
        plsc.subcore_barrier()

        pl.when(c == 0)(lambda: _write_out(s, acc, out0))
        pl.when(c == 1)(lambda: _write_out(s, acc, out1))

    return agg


def _make_agg_fsplit():
    @functools.partial(
        pl.kernel,
        out_type=(
            jax.ShapeDtypeStruct((N, D), jnp.float32),
            jax.ShapeDtypeStruct((N, D), jnp.float32),
        ),
        mesh=_sc_mesh(),
        scratch_types=_AGG_SCRATCH,
    )
    def agg(x0, x1, src2d, dst2d, zrows, out0, out1, srcb, dstb, rows, acc,
            isem, gsem, ssem):
        c = lax.axis_index("c")
        s = lax.axis_index("s")

        _zero_acc(s, zrows, acc)
        nck = jnp.clip(ROWS2D - s * CPT, 0, CPT)

        plsc.subcore_barrier()

        def run(xh, outh):
            _agg_loop(nck, s * CPT, xh, src2d, dst2d, srcb, dstb, rows, acc,
                      isem, gsem, ssem)
            plsc.subcore_barrier()
            _write_out(s, acc, outh)

        pl.when(c == 0)(lambda: run(x0, out0))
        pl.when(c == 1)(lambda: run(x1, out1))

    return agg


def _make_deg():
    @functools.partial(
        pl.kernel,
        out_type=jax.ShapeDtypeStruct((NC * NS * NP,), jnp.float32),
        mesh=_sc_mesh(),
        scratch_types=[
            pltpu.VMEM((CPT, EC), jnp.int32),
            pltpu.VMEM((NP,), jnp.float32),
            pltpu.SemaphoreType.DMA((2,)),
        ],
        compiler_params=pltpu.CompilerParams(needs_layout_passes=False),
    )
    def deg(idx2d, zflat, out, idxb, counts, dsem):
        c = lax.axis_index("c")
        s = lax.axis_index("s")
        w = c * NS + s

        pltpu.async_copy(zflat, counts, dsem.at[0]).wait()
        pltpu.async_copy(
            idx2d.at[pl.ds(c * ROWSPAD + s * CPT, CPT)], idxb, dsem.at[1]
        ).wait()
        nck = jnp.clip(ROWS2D - s * CPT, 0, CPT)

        ones16 = jnp.full((16,), 1.0, jnp.float32)

        def body(k, carry):
            for j in range(EC // 16):
                v = idxb[k, pl.ds(j * 16, 16)]
                plsc.addupdate_scatter(counts, [v], ones16)
            return carry

        lax.fori_loop(0, nck, body, None)

        pltpu.async_copy(counts, out.at[pl.ds(w * NP, NP)], dsem.at[0]).wait()

    return deg


def _dot(a, b):
    return jnp.dot(a, b, preferred_element_type=jnp.float32,
                   precision=lax.Precision.HIGHEST)


def _tc_call(body, out_shapes):
    return pl.pallas_call(
        body,
        out_shape=tuple(jax.ShapeDtypeStruct(s, jnp.float32) for s in out_shapes),
    )


BR = 1000


def _rb_spec(shape):
    if shape[0] == N:
        nd = len(shape)
        return pl.BlockSpec((BR,) + shape[1:], lambda i: (i,) + (0,) * (nd - 1))
    return pl.BlockSpec(shape, lambda i: (0,) * len(shape))


def _tc_rowblock(body, in_shapes, out_shapes):
    return pl.pallas_call(
        body,
        grid=(N // BR,),
        in_specs=[_rb_spec(s) for s in in_shapes],
        out_specs=tuple(_rb_spec(s) for s in out_shapes),
        out_shape=tuple(jax.ShapeDtypeStruct(s, jnp.float32) for s in out_shapes),
    )


def _degsum_body(csr, cdr, gs_r, gd_r):
    gs_r[...] = jnp.sum(csr[...], axis=0)
    gd_r[...] = jnp.sum(cdr[...], axis=0)


def _prep_body(dsr, ddr, xr, inv_s_r, inv_d_r, xs_r):
    ds = dsr[...]
    dd = ddr[...]
    inv_s = jnp.where(ds > 0, lax.rsqrt(jnp.maximum(ds, 1.0)), 0.0)
    inv_d = jnp.where(dd > 0, lax.rsqrt(jnp.maximum(dd, 1.0)), 0.0)
    inv_s_r[...] = inv_s
    inv_d_r[...] = inv_d
    xs_r[...] = xr[...] * inv_s


def _mix1_body(p0r, p1r, invdr, invsr, wr, g0r, g1r):
    y = (p0r[...] + p1r[...]) * invdr[...]
    g = jnp.tanh(_dot(y, wr[...])) * invsr[...]
    g0r[...] = g[:, :D]
    g1r[...] = g[:, D:]


def _mix2_body(y0r, y1r, invdr, invsr, w2r, wcr, qr):
    y = jnp.concatenate([y0r[...], y1r[...]], axis=1) * invdr[...]
    z = jnp.tanh(_dot(y, w2r[...])) * invsr[...]
    qr[...] = _dot(z, wcr[...])


def _fin_body(y0r, y1r, invdr, zmr, zvr):
    yw = (y0r[...] + y1r[...]) * invdr[...]
    p = yw[:, :LD]
    nrm = jnp.sqrt(jnp.sum(p * p, axis=1, keepdims=True))
    zmr[...] = p / (1e-4 + nrm)
    v = yw[:, LD:LD + 1]
    zvr[...] = jnp.log1p(jnp.exp(-jnp.abs(v))) + jnp.maximum(v, 0.0) + 1.0


@jax.jit
def kernel(x, edge_index, Ws1, Ws2, Wm, Ws):
    ei = edge_index.astype(jnp.int32)
    pad = jnp.zeros((2, ROWSPAD * EC - E), jnp.int32)
    ei = jnp.concatenate([ei, pad], axis=1)
    src2d = ei[0].reshape(ROWSPAD, EC)
    dst2d = ei[1].reshape(ROWSPAD, EC)

    w1cat = jnp.concatenate(Ws1, axis=1)
    w2bd = jnp.zeros((R * H, R * H), jnp.float32)
    for r in range(R):
        w2bd = w2bd.at[r * H:(r + 1) * H, r * H:(r + 1) * H].set(Ws2[r])
    wcat = jnp.zeros((R * H, D), jnp.float32)
    wcat = wcat.at[:, :LD].set(Wm).at[:, LD].set(Ws[:, 0])

    zflat = jnp.zeros((NP,), jnp.float32)
    z128 = jnp.zeros((ZR, D), jnp.float32)

    idx2d = jnp.concatenate([src2d, dst2d], axis=0)
    cnt = _make_deg()(idx2d, zflat).reshape(NC, NS, NP // D, D)
    grid_s, grid_d = _tc_call(_degsum_body, [(NP // D, D), (NP // D, D)])(
        cnt[0], cnt[1]
    )
    deg_s = grid_s.reshape(-1)[:N].reshape(N, 1)
    deg_d = grid_d.reshape(-1)[:N].reshape(N, 1)

    inv_s, inv_d, xs = _tc_rowblock(
        _prep_body, [(N, 1), (N, 1), (N, D)], [(N, 1), (N, 1), (N, D)]
    )(deg_s, deg_d, x)

    p0, p1 = _make_agg_esplit()(xs, src2d, dst2d, z128)

    g0, g1 = _tc_rowblock(
        _mix1_body,
        [(N, D), (N, D), (N, 1), (N, 1), (D, R * H)],
        [(N, D), (N, D)],
    )(p0, p1, inv_d, inv_s, w1cat)

    y20, y21 = _make_agg_fsplit()(g0, g1, src2d, dst2d, z128)

    qp = _tc_rowblock(
        _mix2_body,
        [(N, D), (N, D), (N, 1), (N, 1), (R * H, R * H), (R * H, D)],
        [(N, D)],
    )(y20, y21, inv_d, inv_s, w2bd, wcat)[0]

    y3a, y3b = _make_agg_esplit()(qp, src2d, dst2d, z128)

    z_mean, z_var = _tc_rowblock(
        _fin_body,
        [(N, D), (N, D), (N, 1)],
        [(N, LD), (N, 1)],
    )(y3a, y3b, inv_d)

    return z_mean, z_mean, z_var


# speedup vs baseline: 24.1274x; 1.0446x over previous
import functools

import jax
import jax.numpy as jnp
from jax import lax
from jax.experimental import pallas as pl
from jax.experimental.pallas import tpu as pltpu
from jax.experimental.pallas import tpu_sc as plsc

N = 10000
E = 320000
D = 128
R = 4
H = 64
LD = 32

NC = 2
NS = 16
EC = 128
ROWS2D = E // EC
CPT = 160
CPT2 = 80
ROWSPAD = NS * CPT
ZR = 200
NZC = N // ZR
NP = 10240


def _sc_mesh():
    return plsc.VectorSubcoreMesh(
        core_axis_name="c", subcore_axis_name="s", num_cores=NC, num_subcores=NS
    )


def _zero_acc(s, zrows, acc):
    for j in range(4):
        idx = s + NS * j

        @pl.when(idx < NZC)
        def _():
            pltpu.sync_copy(zrows, acc.at[pl.ds(idx * ZR, ZR)])


def _write_out(s, acc, outh):
    for j in range(4):
        idx = s + NS * j

        @pl.when(idx < NZC)
        def _():
            pltpu.sync_copy(acc.at[pl.ds(idx * ZR, ZR)], outh.at[pl.ds(idx * ZR, ZR)])


G = 32


def _agg_loop(nck, tilebase, xh, src2d, dst2d, srcb, dstb, rows, acc,
              isem, gsem, ssem):
    def idx_start(g, p):
        base = tilebase + g * G
        pltpu.async_copy(src2d.at[pl.ds(base, G)], srcb.at[p], isem.at[p])
        pltpu.async_copy(dst2d.at[pl.ds(base, G)], dstb.at[p], isem.at[p])

    def idx_wait(g, p):
        base = tilebase + g * G
        pltpu.make_async_copy(src2d.at[pl.ds(base, G)], srcb.at[p], isem.at[p]).wait()
        pltpu.make_async_copy(dst2d.at[pl.ds(base, G)], dstb.at[p], isem.at[p]).wait()

    def g_start(k, p, j, b):
        pltpu.async_copy(xh.at[srcb.at[p, j]], rows.at[b], gsem.at[b])

    def g_wait(k, p, j, b):
        pltpu.make_async_copy(xh.at[srcb.at[p, j]], rows.at[b], gsem.at[b]).wait()

    idx_start(0, 0)

    def body(k, _):
        j = jnp.bitwise_and(k, G - 1)
        g = lax.shift_right_logical(k, 5)
        p = jnp.bitwise_and(g, 1)
        b = jnp.bitwise_and(k, 1)

        @pl.when(j == 0)
        def _():
            idx_wait(g, p)
            pl.when((g + 1) * G < nck)(lambda: idx_start(g + 1, 1 - p))
            g_start(k, p, 0, b)
            pl.when(k + 1 < nck)(lambda: g_start(k + 1, p, 1, 1 - b))

        g_wait(k, p, j, b)
        pltpu.async_copy(rows.at[b], acc.at[dstb.at[p, j]], ssem.at[b], add=True)
        pltpu.make_async_copy(rows.at[b], acc.at[dstb.at[p, j]], ssem.at[b]).wait()
        pl.when(jnp.logical_and(j < G - 2, k + 2 < nck))(
            lambda: g_start(k + 2, p, j + 2, b)
        )
        return _

    lax.fori_loop(0, nck, body, None)


_AGG_SCRATCH = [
    pltpu.VMEM((2, G, EC), jnp.int32),
    pltpu.VMEM((2, G, EC), jnp.int32),
    pltpu.VMEM((2, EC, D), jnp.float32),
    pltpu.VMEM_SHARED((N, D), jnp.float32),
    pltpu.SemaphoreType.DMA((2,)),
    pltpu.SemaphoreType.DMA((2,)),
    pltpu.SemaphoreType.DMA((2,)),
]


def _make_agg_esplit():
    @functools.partial(
        pl.kernel,
        out_type=(
            jax.ShapeDtypeStruct((N, D), jnp.float32),
            jax.ShapeDtypeStruct((N, D), jnp.float32),
        ),
        mesh=_sc_mesh(),
        scratch_types=_AGG_SCRATCH,
    )
    def agg(x, src2d, dst2d, zrows, out0, out1, srcb, dstb, rows, acc, isem, gsem, ssem):
        c = lax.axis_index("c")
        s = lax.axis_index("s")
        w = c * NS + s

        _zero_acc(s, zrows, acc)
        nck = jnp.clip(ROWS2D - w * CPT2, 0, CPT2)

        plsc.subcore_barrier()
        _agg_loop(nck, w * CPT2, x, src2d, dst2d, srcb, dstb, rows, acc,
                  isem, gsem, ssem)
        plsc.subcore_barrier()

        pl.when(c == 0)(lambda: _write_out(s, acc, out0))
        pl.when(c == 1)(lambda: _write_out(s, acc, out1))

    return agg


def _make_agg_fsplit():
    @functools.partial(
        pl.kernel,
        out_type=(
            jax.ShapeDtypeStruct((N, D), jnp.float32),
            jax.ShapeDtypeStruct((N, D), jnp.float32),
        ),
        mesh=_sc_mesh(),
        scratch_types=_AGG_SCRATCH,
    )
    def agg(x0, x1, src2d, dst2d, zrows, out0, out1, srcb, dstb, rows, acc,
            isem, gsem, ssem):
        c = lax.axis_index("c")
        s = lax.axis_index("s")

        _zero_acc(s, zrows, acc)
        nck = jnp.clip(ROWS2D - s * CPT, 0, CPT)

        plsc.subcore_barrier()

        def run(xh, outh):
            _agg_loop(nck, s * CPT, xh, src2d, dst2d, srcb, dstb, rows, acc,
                      isem, gsem, ssem)
            plsc.subcore_barrier()
            _write_out(s, acc, outh)

        pl.when(c == 0)(lambda: run(x0, out0))
        pl.when(c == 1)(lambda: run(x1, out1))

    return agg


def _make_deg():
    @functools.partial(
        pl.kernel,
        out_type=jax.ShapeDtypeStruct((NC * NS * NP,), jnp.float32),
        mesh=_sc_mesh(),
        scratch_types=[
            pltpu.VMEM((CPT, EC), jnp.int32),
            pltpu.VMEM((NP,), jnp.float32),
            pltpu.SemaphoreType.DMA((2,)),
        ],
        compiler_params=pltpu.CompilerParams(needs_layout_passes=False),
    )
    def deg(idx2d, zflat, out, idxb, counts, dsem):
        c = lax.axis_index("c")
        s = lax.axis_index("s")
        w = c * NS + s

        pltpu.async_copy(zflat, counts, dsem.at[0]).wait()
        pltpu.async_copy(
            idx2d.at[pl.ds(c * ROWSPAD + s * CPT, CPT)], idxb, dsem.at[1]
        ).wait()
        nck = jnp.clip(ROWS2D - s * CPT, 0, CPT)

        ones16 = jnp.full((16,), 1.0, jnp.float32)

        def body(k, carry):
            for j in range(EC // 16):
                v = idxb[k, pl.ds(j * 16, 16)]
                plsc.addupdate_scatter(counts, [v], ones16)
            return carry

        lax.fori_loop(0, nck, body, None)

        pltpu.async_copy(counts, out.at[pl.ds(w * NP, NP)], dsem.at[0]).wait()

    return deg


def _dot(a, b):
    return jnp.dot(a, b, preferred_element_type=jnp.float32)


def _tc_call(body, out_shapes):
    return pl.pallas_call(
        body,
        out_shape=tuple(jax.ShapeDtypeStruct(s, jnp.float32) for s in out_shapes),
    )


BR = 1000


def _rb_spec(shape):
    if shape[0] == N:
        nd = len(shape)
        return pl.BlockSpec((BR,) + shape[1:], lambda i: (i,) + (0,) * (nd - 1))
    return pl.BlockSpec(shape, lambda i: (0,) * len(shape))


def _tc_rowblock(body, in_shapes, out_shapes):
    return pl.pallas_call(
        body,
        grid=(N // BR,),
        in_specs=[_rb_spec(s) for s in in_shapes],
        out_specs=tuple(_rb_spec(s) for s in out_shapes),
        out_shape=tuple(jax.ShapeDtypeStruct(s, jnp.float32) for s in out_shapes),
    )


def _degsum_body(csr, cdr, gs_r, gd_r):
    gs_r[...] = jnp.sum(csr[...], axis=0)
    gd_r[...] = jnp.sum(cdr[...], axis=0)


def _prep_body(dsr, ddr, xr, inv_s_r, inv_d_r, xs_r):
    ds = dsr[...]
    dd = ddr[...]
    inv_s = jnp.where(ds > 0, lax.rsqrt(jnp.maximum(ds, 1.0)), 0.0)
    inv_d = jnp.where(dd > 0, lax.rsqrt(jnp.maximum(dd, 1.0)), 0.0)
    inv_s_r[...] = inv_s
    inv_d_r[...] = inv_d
    xs_r[...] = xr[...] * inv_s


def _mix1_body(p0r, p1r, invdr, invsr, wr, g0r, g1r):
    y = (p0r[...] + p1r[...]) * invdr[...]
    g = jnp.tanh(_dot(y, wr[...])) * invsr[...]
    g0r[...] = g[:, :D]
    g1r[...] = g[:, D:]


def _mix2_body(y0r, y1r, invdr, invsr, w2r, wcr, qr):
    y = jnp.concatenate([y0r[...], y1r[...]], axis=1) * invdr[...]
    z = jnp.tanh(_dot(y, w2r[...])) * invsr[...]
    qr[...] = _dot(z, wcr[...])


def _fin_body(y0r, y1r, invdr, zmr, zvr):
    yw = (y0r[...] + y1r[...]) * invdr[...]
    p = yw[:, :LD]
    nrm = jnp.sqrt(jnp.sum(p * p, axis=1, keepdims=True))
    zmr[...] = p / (1e-4 + nrm)
    v = yw[:, LD:LD + 1]
    zvr[...] = jnp.log1p(jnp.exp(-jnp.abs(v))) + jnp.maximum(v, 0.0) + 1.0


@jax.jit
def kernel(x, edge_index, Ws1, Ws2, Wm, Ws):
    ei = edge_index.astype(jnp.int32)
    pad = jnp.zeros((2, ROWSPAD * EC - E), jnp.int32)
    ei = jnp.concatenate([ei, pad], axis=1)
    src2d = ei[0].reshape(ROWSPAD, EC)
    dst2d = ei[1].reshape(ROWSPAD, EC)

    w1cat = jnp.concatenate(Ws1, axis=1)
    w2bd = jnp.zeros((R * H, R * H), jnp.float32)
    for r in range(R):
        w2bd = w2bd.at[r * H:(r + 1) * H, r * H:(r + 1) * H].set(Ws2[r])
    wcat = jnp.zeros((R * H, D), jnp.float32)
    wcat = wcat.at[:, :LD].set(Wm).at[:, LD].set(Ws[:, 0])

    zflat = jnp.zeros((NP,), jnp.float32)
    z128 = jnp.zeros((ZR, D), jnp.float32)

    idx2d = jnp.concatenate([src2d, dst2d], axis=0)
    cnt = _make_deg()(idx2d, zflat).reshape(NC, NS, NP // D, D)
    grid_s, grid_d = _tc_call(_degsum_body, [(NP // D, D), (NP // D, D)])(
        cnt[0], cnt[1]
    )
    deg_s = grid_s.reshape(-1)[:N].reshape(N, 1)
    deg_d = grid_d.reshape(-1)[:N].reshape(N, 1)

    inv_s, inv_d, xs = _tc_rowblock(
        _prep_body, [(N, 1), (N, 1), (N, D)], [(N, 1), (N, 1), (N, D)]
    )(deg_s, deg_d, x)

    p0, p1 = _make_agg_esplit()(xs, src2d, dst2d, z128)

    g0, g1 = _tc_rowblock(
        _mix1_body,
        [(N, D), (N, D), (N, 1), (N, 1), (D, R * H)],
        [(N, D), (N, D)],
    )(p0, p1, inv_d, inv_s, w1cat)

    y20, y21 = _make_agg_fsplit()(g0, g1, src2d, dst2d, z128)

    qp = _tc_rowblock(
        _mix2_body,
        [(N, D), (N, D), (N, 1), (N, 1), (R * H, R * H), (R * H, D)],
        [(N, D)],
    )(y20, y21, inv_d, inv_s, w2bd, wcat)[0]

    y3a, y3b = _make_agg_esplit()(qp, src2d, dst2d, z128)

    z_mean, z_var = _tc_rowblock(
        _fin_body,
        [(N, D), (N, D), (N, 1)],
        [(N, LD), (N, 1)],
    )(y3a, y3b, inv_d)

    return z_mean, z_mean, z_var


# speedup vs baseline: 25.4742x; 1.0558x over previous
import functools

import jax
import jax.numpy as jnp
from jax import lax
from jax.experimental import pallas as pl
from jax.experimental.pallas import tpu as pltpu
from jax.experimental.pallas import tpu_sc as plsc

N = 10000
E = 320000
D = 128
R = 4
H = 64
LD = 32

NC = 2
NS = 16
EC = 128
ROWS2D = E // EC
CPT = 160
CPT2 = 80
ROWSPAD = NS * CPT
ZR = 200
NZC = N // ZR
NP = 10240


def _sc_mesh():
    return plsc.VectorSubcoreMesh(
        core_axis_name="c", subcore_axis_name="s", num_cores=NC, num_subcores=NS
    )


def _zero_acc(s, zrows, acc):
    for j in range(4):
        idx = s + NS * j

        @pl.when(idx < NZC)
        def _():
            pltpu.sync_copy(zrows, acc.at[pl.ds(idx * ZR, ZR)])


def _write_out(s, acc, outh):
    for j in range(4):
        idx = s + NS * j

        @pl.when(idx < NZC)
        def _():
            pltpu.sync_copy(acc.at[pl.ds(idx * ZR, ZR)], outh.at[pl.ds(idx * ZR, ZR)])


G = 32


def _agg_loop(nck, tilebase, xh, src2d, dst2d, srcb, dstb, rows, acc,
              isem, gsem, ssem):
    def idx_start(g, p):
        base = tilebase + g * G
        pltpu.async_copy(src2d.at[pl.ds(base, G)], srcb.at[p], isem.at[p])
        pltpu.async_copy(dst2d.at[pl.ds(base, G)], dstb.at[p], isem.at[p])

    def idx_wait(g, p):
        base = tilebase + g * G
        pltpu.make_async_copy(src2d.at[pl.ds(base, G)], srcb.at[p], isem.at[p]).wait()
        pltpu.make_async_copy(dst2d.at[pl.ds(base, G)], dstb.at[p], isem.at[p]).wait()

    def g_start(k, p, j, b):
        pltpu.async_copy(xh.at[srcb.at[p, j]], rows.at[b], gsem.at[b])

    def g_wait(k, p, j, b):
        pltpu.make_async_copy(xh.at[srcb.at[p, j]], rows.at[b], gsem.at[b]).wait()

    idx_start(0, 0)

    def body(k, _):
        j = jnp.bitwise_and(k, G - 1)
        g = lax.shift_right_logical(k, 5)
        p = jnp.bitwise_and(g, 1)
        b = jnp.bitwise_and(k, 1)

        @pl.when(j == 0)
        def _():
            idx_wait(g, p)
            pl.when((g + 1) * G < nck)(lambda: idx_start(g + 1, 1 - p))
            g_start(k, p, 0, b)
            pl.when(k + 1 < nck)(lambda: g_start(k + 1, p, 1, 1 - b))

        g_wait(k, p, j, b)
        pltpu.async_copy(rows.at[b], acc.at[dstb.at[p, j]], ssem.at[b], add=True)
        pltpu.make_async_copy(rows.at[b], acc.at[dstb.at[p, j]], ssem.at[b]).wait()
        pl.when(jnp.logical_and(j < G - 2, k + 2 < nck))(
            lambda: g_start(k + 2, p, j + 2, b)
        )
        return _

    lax.fori_loop(0, nck, body, None)


def _agg_scratch(fw):
    return [
        pltpu.VMEM((2, G, EC), jnp.int32),
        pltpu.VMEM((2, G, EC), jnp.int32),
        pltpu.VMEM((2, EC, fw), jnp.float32),
        pltpu.VMEM_SHARED((N, fw), jnp.float32),
        pltpu.SemaphoreType.DMA((2,)),
        pltpu.SemaphoreType.DMA((2,)),
        pltpu.SemaphoreType.DMA((2,)),
    ]


def _make_agg_esplit(fw=D, tc_tiling=True):
    @functools.partial(
        pl.kernel,
        out_type=(
            jax.ShapeDtypeStruct((N, fw), jnp.float32),
            jax.ShapeDtypeStruct((N, fw), jnp.float32),
        ),
        mesh=_sc_mesh(),
        scratch_types=_agg_scratch(fw),
        compiler_params=pltpu.CompilerParams(use_tc_tiling_on_sc=tc_tiling),
    )
    def agg(x, src2d, dst2d, zrows, out0, out1, srcb, dstb, rows, acc, isem, gsem, ssem):
        c = lax.axis_index("c")
        s = lax.axis_index("s")
        w = c * NS + s

        _zero_acc(s, zrows, acc)
        nck = jnp.clip(ROWS2D - w * CPT2, 0, CPT2)

        plsc.subcore_barrier()
        _agg_loop(nck, w * CPT2, x, src2d, dst2d, srcb, dstb, rows, acc,
                  isem, gsem, ssem)
        plsc.subcore_barrier()

        pl.when(c == 0)(lambda: _write_out(s, acc, out0))
        pl.when(c == 1)(lambda: _write_out(s, acc, out1))

    return agg


def _make_agg_fsplit():
    @functools.partial(
        pl.kernel,
        out_type=(
            jax.ShapeDtypeStruct((N, D), jnp.float32),
            jax.ShapeDtypeStruct((N, D), jnp.float32),
        ),
        mesh=_sc_mesh(),
        scratch_types=_agg_scratch(D),
    )
    def agg(x0, x1, src2d, dst2d, zrows, out0, out1, srcb, dstb, rows, acc,
            isem, gsem, ssem):
        c = lax.axis_index("c")
        s = lax.axis_index("s")

        _zero_acc(s, zrows, acc)
        nck = jnp.clip(ROWS2D - s * CPT, 0, CPT)

        plsc.subcore_barrier()

        def run(xh, outh):
            _agg_loop(nck, s * CPT, xh, src2d, dst2d, srcb, dstb, rows, acc,
                      isem, gsem, ssem)
            plsc.subcore_barrier()
            _write_out(s, acc, outh)

        pl.when(c == 0)(lambda: run(x0, out0))
        pl.when(c == 1)(lambda: run(x1, out1))

    return agg


def _make_deg():
    @functools.partial(
        pl.kernel,
        out_type=jax.ShapeDtypeStruct((NC * NS * NP,), jnp.float32),
        mesh=_sc_mesh(),
        scratch_types=[
            pltpu.VMEM((CPT, EC), jnp.int32),
            pltpu.VMEM((NP,), jnp.float32),
            pltpu.SemaphoreType.DMA((2,)),
        ],
        compiler_params=pltpu.CompilerParams(needs_layout_passes=False),
    )
    def deg(idx2d, zflat, out, idxb, counts, dsem):
        c = lax.axis_index("c")
        s = lax.axis_index("s")
        w = c * NS + s

        pltpu.async_copy(zflat, counts, dsem.at[0]).wait()
        pltpu.async_copy(
            idx2d.at[pl.ds(c * ROWSPAD + s * CPT, CPT)], idxb, dsem.at[1]
        ).wait()
        nck = jnp.clip(ROWS2D - s * CPT, 0, CPT)

        ones16 = jnp.full((16,), 1.0, jnp.float32)

        def body(k, carry):
            for j in range(EC // 16):
                v = idxb[k, pl.ds(j * 16, 16)]
                plsc.addupdate_scatter(counts, [v], ones16)
            return carry

        lax.fori_loop(0, nck, body, None)

        pltpu.async_copy(counts, out.at[pl.ds(w * NP, NP)], dsem.at[0]).wait()

    return deg


def _dot(a, b):
    return jnp.dot(a, b, preferred_element_type=jnp.float32)


def _tc_call(body, out_shapes):
    return pl.pallas_call(
        body,
        out_shape=tuple(jax.ShapeDtypeStruct(s, jnp.float32) for s in out_shapes),
    )


BR = 1000


def _rb_spec(shape):
    if shape[0] == N:
        nd = len(shape)
        return pl.BlockSpec((BR,) + shape[1:], lambda i: (i,) + (0,) * (nd - 1))
    return pl.BlockSpec(shape, lambda i: (0,) * len(shape))


def _tc_rowblock(body, in_shapes, out_shapes):
    return pl.pallas_call(
        body,
        grid=(N // BR,),
        in_specs=[_rb_spec(s) for s in in_shapes],
        out_specs=tuple(_rb_spec(s) for s in out_shapes),
        out_shape=tuple(jax.ShapeDtypeStruct(s, jnp.float32) for s in out_shapes),
    )


def _degsum_body(csr, cdr, gs_r, gd_r):
    gs_r[...] = jnp.sum(csr[...], axis=0)
    gd_r[...] = jnp.sum(cdr[...], axis=0)


def _prep_body(dsr, ddr, xr, inv_s_r, inv_d_r, xs_r):
    ds = dsr[...]
    dd = ddr[...]
    inv_s = jnp.where(ds > 0, lax.rsqrt(jnp.maximum(ds, 1.0)), 0.0)
    inv_d = jnp.where(dd > 0, lax.rsqrt(jnp.maximum(dd, 1.0)), 0.0)
    inv_s_r[...] = inv_s
    inv_d_r[...] = inv_d
    xs_r[...] = xr[...] * inv_s


def _mix1_body(p0r, p1r, invdr, invsr, wr, g0r, g1r):
    y = (p0r[...] + p1r[...]) * invdr[...]
    g = jnp.tanh(_dot(y, wr[...])) * invsr[...]
    g0r[...] = g[:, :D]
    g1r[...] = g[:, D:]


def _mix2_body(y0r, y1r, invdr, invsr, w2r, wcr, qr):
    y = jnp.concatenate([y0r[...], y1r[...]], axis=1) * invdr[...]
    z = jnp.tanh(_dot(y, w2r[...])) * invsr[...]
    qr[...] = _dot(z, wcr[...])


def _fin_body(y0r, y1r, invdr, zmr, zvr):
    yw = (y0r[...] + y1r[...]) * invdr[...]
    p = yw[:, :LD]
    nrm = jnp.sqrt(jnp.sum(p * p, axis=1, keepdims=True))
    zmr[...] = p / (1e-4 + nrm)
    v = yw[:, LD:LD + 1]
    zvr[...] = jnp.log1p(jnp.exp(-jnp.abs(v))) + jnp.maximum(v, 0.0) + 1.0


@jax.jit
def kernel(x, edge_index, Ws1, Ws2, Wm, Ws):
    ei = edge_index.astype(jnp.int32)
    pad = jnp.zeros((2, ROWSPAD * EC - E), jnp.int32)
    ei = jnp.concatenate([ei, pad], axis=1)
    src2d = ei[0].reshape(ROWSPAD, EC)
    dst2d = ei[1].reshape(ROWSPAD, EC)

    w1cat = jnp.concatenate(Ws1, axis=1)
    w2bd = jnp.zeros((R * H, R * H), jnp.float32)
    for r in range(R):
        w2bd = w2bd.at[r * H:(r + 1) * H, r * H:(r + 1) * H].set(Ws2[r])
    wcat = jnp.zeros((R * H, 48), jnp.float32)
    wcat = wcat.at[:, :LD].set(Wm).at[:, LD].set(Ws[:, 0])

    zflat = jnp.zeros((NP,), jnp.float32)
    z128 = jnp.zeros((ZR, D), jnp.float32)
    z48 = jnp.zeros((ZR, 48), jnp.float32)

    idx2d = jnp.concatenate([src2d, dst2d], axis=0)
    cnt = _make_deg()(idx2d, zflat).reshape(NC, NS, NP // D, D)
    grid_s, grid_d = _tc_call(_degsum_body, [(NP // D, D), (NP // D, D)])(
        cnt[0], cnt[1]
    )
    deg_s = grid_s.reshape(-1)[:N].reshape(N, 1)
    deg_d = grid_d.reshape(-1)[:N].reshape(N, 1)

    inv_s, inv_d, xs = _tc_rowblock(
        _prep_body, [(N, 1), (N, 1), (N, D)], [(N, 1), (N, 1), (N, D)]
    )(deg_s, deg_d, x)

    p0, p1 = _make_agg_esplit()(xs, src2d, dst2d, z128)

    g0, g1 = _tc_rowblock(
        _mix1_body,
        [(N, D), (N, D), (N, 1), (N, 1), (D, R * H)],
        [(N, D), (N, D)],
    )(p0, p1, inv_d, inv_s, w1cat)

    y20, y21 = _make_agg_fsplit()(g0, g1, src2d, dst2d, z128)

    qp = _tc_rowblock(
        _mix2_body,
        [(N, D), (N, D), (N, 1), (N, 1), (R * H, R * H), (R * H, 48)],
        [(N, 48)],
    )(y20, y21, inv_d, inv_s, w2bd, wcat)[0]

    y3a, y3b = _make_agg_esplit(48, False)(qp, src2d, dst2d, z48)

    z_mean, z_var = _tc_rowblock(
        _fin_body,
        [(N, 48), (N, 48), (N, 1)],
        [(N, LD), (N, 1)],
    )(y3a, y3b, inv_d)

    return z_mean, z_mean, z_var


# speedup vs baseline: 26.7193x; 1.0489x over previous
import functools

import jax
import jax.numpy as jnp
from jax import lax
from jax.experimental import pallas as pl
from jax.experimental.pallas import tpu as pltpu
from jax.experimental.pallas import tpu_sc as plsc

N = 10000
E = 320000
D = 128
R = 4
H = 64
LD = 32

NC = 2
NS = 16
ECA = 100
RA = E // ECA
CPTA = 208
CPTA2 = 104
RPA = NS * CPTA
GA = 8
ECD = 128
RD = E // ECD
CPTD = 160
RPD = NS * CPTD
ZR = 200
NZC = N // ZR
NP = 10240


def _sc_mesh():
    return plsc.VectorSubcoreMesh(
        core_axis_name="c", subcore_axis_name="s", num_cores=NC, num_subcores=NS
    )


def _zero_acc(s, zrows, acc):
    for j in range(4):
        idx = s + NS * j

        @pl.when(idx < NZC)
        def _():
            pltpu.sync_copy(zrows, acc.at[pl.ds(idx * ZR, ZR)])


def _write_out(s, acc, outh):
    for j in range(4):
        idx = s + NS * j

        @pl.when(idx < NZC)
        def _():
            pltpu.sync_copy(acc.at[pl.ds(idx * ZR, ZR)], outh.at[pl.ds(idx * ZR, ZR)])


def _agg_loop(nck, tilebase, xh, src2d, dst2d, srcb, dstb, rows, acc,
              isem, gsem, ssem):
    def idx_start(g, p):
        base = tilebase + g * GA
        pltpu.async_copy(src2d.at[pl.ds(base, GA)], srcb.at[p], isem.at[p])
        pltpu.async_copy(dst2d.at[pl.ds(base, GA)], dstb.at[p], isem.at[p])

    def idx_wait(g, p):
        base = tilebase + g * GA
        pltpu.make_async_copy(src2d.at[pl.ds(base, GA)], srcb.at[p], isem.at[p]).wait()
        pltpu.make_async_copy(dst2d.at[pl.ds(base, GA)], dstb.at[p], isem.at[p]).wait()

    def g_start(m):
        p = jnp.bitwise_and(lax.shift_right_logical(m, 3), 1)
        j = jnp.bitwise_and(m, GA - 1)
        b = lax.rem(m, 3)
        pltpu.async_copy(xh.at[srcb.at[p, j]], rows.at[b], gsem.at[b])

    def g_wait(m):
        p = jnp.bitwise_and(lax.shift_right_logical(m, 3), 1)
        j = jnp.bitwise_and(m, GA - 1)
        b = lax.rem(m, 3)
        pltpu.make_async_copy(xh.at[srcb.at[p, j]], rows.at[b], gsem.at[b]).wait()

    def s_start(m):
        p = jnp.bitwise_and(lax.shift_right_logical(m, 3), 1)
        j = jnp.bitwise_and(m, GA - 1)
        b = lax.rem(m, 3)
        pltpu.async_copy(rows.at[b], acc.at[dstb.at[p, j]], ssem.at[b], add=True)

    def s_wait(m):
        p = jnp.bitwise_and(lax.shift_right_logical(m, 3), 1)
        j = jnp.bitwise_and(m, GA - 1)
        b = lax.rem(m, 3)
        pltpu.make_async_copy(rows.at[b], acc.at[dstb.at[p, j]], ssem.at[b]).wait()

    idx_start(0, 0)
    idx_wait(0, 0)
    pl.when(nck > 0)(lambda: g_start(0))

    def body(k, carry):
        j = jnp.bitwise_and(k, GA - 1)
        g = lax.shift_right_logical(k, 3)
        p = jnp.bitwise_and(g, 1)

        pl.when(jnp.logical_and(j == 2, (g + 1) * GA < nck))(
            lambda: idx_start(g + 1, 1 - p)
        )
        pl.when(jnp.logical_and(j == GA - 1, k + 1 < nck))(
            lambda: idx_wait(g + 1, 1 - p)
        )

        @pl.when(k + 1 < nck)
        def _():
            pl.when(k >= 2)(lambda: s_wait(k - 2))
            g_start(k + 1)

        g_wait(k)
        s_start(k)
        return carry

    lax.fori_loop(0, nck, body, None)
    for d in (3, 2, 1):
        pl.when(nck >= d)(lambda d=d: s_wait(nck - d))


def _agg_scratch(fw):
    return [
        pltpu.VMEM((2, GA, ECA), jnp.int32),
        pltpu.VMEM((2, GA, ECA), jnp.int32),
        pltpu.VMEM((3, ECA, fw), jnp.float32),
        pltpu.VMEM_SHARED((N, fw), jnp.float32),
        pltpu.SemaphoreType.DMA((2,)),
        pltpu.SemaphoreType.DMA((3,)),
        pltpu.SemaphoreType.DMA((3,)),
    ]


def _make_agg_esplit(fw=D, tc_tiling=True):
    @functools.partial(
        pl.kernel,
        out_type=(
            jax.ShapeDtypeStruct((N, fw), jnp.float32),
            jax.ShapeDtypeStruct((N, fw), jnp.float32),
        ),
        mesh=_sc_mesh(),
        scratch_types=_agg_scratch(fw),
        compiler_params=pltpu.CompilerParams(use_tc_tiling_on_sc=tc_tiling),
    )
    def agg(x, src2d, dst2d, zrows, out0, out1, srcb, dstb, rows, acc, isem, gsem, ssem):
        c = lax.axis_index("c")
        s = lax.axis_index("s")
        w = c * NS + s

        _zero_acc(s, zrows, acc)
        nck = jnp.clip(RA - w * CPTA2, 0, CPTA2)

        plsc.subcore_barrier()
        _agg_loop(nck, w * CPTA2, x, src2d, dst2d, srcb, dstb, rows, acc,
                  isem, gsem, ssem)
        plsc.subcore_barrier()

        pl.when(c == 0)(lambda: _write_out(s, acc, out0))
        pl.when(c == 1)(lambda: _write_out(s, acc, out1))

    return agg


def _make_agg_fsplit():
    @functools.partial(
        pl.kernel,
        out_type=(
            jax.ShapeDtypeStruct((N, D), jnp.float32),
            jax.ShapeDtypeStruct((N, D), jnp.float32),
        ),
        mesh=_sc_mesh(),
        scratch_types=_agg_scratch(D),
    )
    def agg(x0, x1, src2d, dst2d, zrows, out0, out1, srcb, dstb, rows, acc,
            isem, gsem, ssem):
        c = lax.axis_index("c")
        s = lax.axis_index("s")

        _zero_acc(s, zrows, acc)
        nck = jnp.clip(RA - s * CPTA, 0, CPTA)

        plsc.subcore_barrier()

        def run(xh, outh):
            _agg_loop(nck, s * CPTA, xh, src2d, dst2d, srcb, dstb, rows, acc,
                      isem, gsem, ssem)
            plsc.subcore_barrier()
            _write_out(s, acc, outh)

        pl.when(c == 0)(lambda: run(x0, out0))
        pl.when(c == 1)(lambda: run(x1, out1))

    return agg


def _make_deg():
    @functools.partial(
        pl.kernel,
        out_type=jax.ShapeDtypeStruct((NC * NS * NP,), jnp.float32),
        mesh=_sc_mesh(),
        scratch_types=[
            pltpu.VMEM((CPTD, ECD), jnp.int32),
            pltpu.VMEM((NP,), jnp.float32),
            pltpu.SemaphoreType.DMA((2,)),
        ],
        compiler_params=pltpu.CompilerParams(needs_layout_passes=False),
    )
    def deg(idx2d, zflat, out, idxb, counts, dsem):
        c = lax.axis_index("c")
        s = lax.axis_index("s")
        w = c * NS + s

        pltpu.async_copy(zflat, counts, dsem.at[0]).wait()
        pltpu.async_copy(
            idx2d.at[pl.ds(c * RPD + s * CPTD, CPTD)], idxb, dsem.at[1]
        ).wait()
        nck = jnp.clip(RD - s * CPTD, 0, CPTD)

        ones16 = jnp.full((16,), 1.0, jnp.float32)

        def body(k, carry):
            for j in range(ECD // 16):
                v = idxb[k, pl.ds(j * 16, 16)]
                plsc.addupdate_scatter(counts, [v], ones16)
            return carry

        lax.fori_loop(0, nck, body, None)

        pltpu.async_copy(counts, out.at[pl.ds(w * NP, NP)], dsem.at[0]).wait()

    return deg


def _dot(a, b):
    return jnp.dot(a, b, preferred_element_type=jnp.float32)


def _tc_call(body, out_shapes):
    return pl.pallas_call(
        body,
        out_shape=tuple(jax.ShapeDtypeStruct(s, jnp.float32) for s in out_shapes),
    )


BR = 1000


def _rb_spec(shape):
    if shape[0] == N:
        nd = len(shape)
        return pl.BlockSpec((BR,) + shape[1:], lambda i: (i,) + (0,) * (nd - 1))
    return pl.BlockSpec(shape, lambda i: (0,) * len(shape))


def _tc_rowblock(body, in_shapes, out_shapes):
    return pl.pallas_call(
        body,
        grid=(N // BR,),
        in_specs=[_rb_spec(s) for s in in_shapes],
        out_specs=tuple(_rb_spec(s) for s in out_shapes),
        out_shape=tuple(jax.ShapeDtypeStruct(s, jnp.float32) for s in out_shapes),
    )


def _degsum_body(csr, cdr, gs_r, gd_r):
    gs_r[...] = jnp.sum(csr[...], axis=0)
    gd_r[...] = jnp.sum(cdr[...], axis=0)


def _prep_body(dsr, ddr, xr, inv_s_r, inv_d_r, xs_r):
    ds = dsr[...]
    dd = ddr[...]
    inv_s = jnp.where(ds > 0, lax.rsqrt(jnp.maximum(ds, 1.0)), 0.0)
    inv_d = jnp.where(dd > 0, lax.rsqrt(jnp.maximum(dd, 1.0)), 0.0)
    inv_s_r[...] = inv_s
    inv_d_r[...] = inv_d
    xs_r[...] = xr[...] * inv_s


def _mix1_body(p0r, p1r, invdr, invsr, wr, g0r, g1r):
    y = (p0r[...] + p1r[...]) * invdr[...]
    g = jnp.tanh(_dot(y, wr[...])) * invsr[...]
    g0r[...] = g[:, :D]
    g1r[...] = g[:, D:]


def _mix2_body(y0r, y1r, invdr, invsr, w2r, wcr, qr):
    y = jnp.concatenate([y0r[...], y1r[...]], axis=1) * invdr[...]
    z = jnp.tanh(_dot(y, w2r[...])) * invsr[...]
    qr[...] = _dot(z, wcr[...])


def _fin_body(y0r, y1r, invdr, zmr, zvr):
    yw = (y0r[...] + y1r[...]) * invdr[...]
    p = yw[:, :LD]
    nrm = jnp.sqrt(jnp.sum(p * p, axis=1, keepdims=True))
    zmr[...] = p / (1e-4 + nrm)
    v = yw[:, LD:LD + 1]
    zvr[...] = jnp.log1p(jnp.exp(-jnp.abs(v))) + jnp.maximum(v, 0.0) + 1.0


@jax.jit
def kernel(x, edge_index, Ws1, Ws2, Wm, Ws):
    ei = edge_index.astype(jnp.int32)
    eia = jnp.concatenate([ei, jnp.zeros((2, RPA * ECA - E), jnp.int32)], axis=1)
    src2d = eia[0].reshape(RPA, ECA)
    dst2d = eia[1].reshape(RPA, ECA)
    eid = jnp.concatenate([ei, jnp.zeros((2, RPD * ECD - E), jnp.int32)], axis=1)
    src2d_d = eid[0].reshape(RPD, ECD)
    dst2d_d = eid[1].reshape(RPD, ECD)

    w1cat = jnp.concatenate(Ws1, axis=1)
    w2bd = jnp.zeros((R * H, R * H), jnp.float32)
    for r in range(R):
        w2bd = w2bd.at[r * H:(r + 1) * H, r * H:(r + 1) * H].set(Ws2[r])
    wcat = jnp.zeros((R * H, 48), jnp.float32)
    wcat = wcat.at[:, :LD].set(Wm).at[:, LD].set(Ws[:, 0])

    zflat = jnp.zeros((NP,), jnp.float32)
    z128 = jnp.zeros((ZR, D), jnp.float32)
    z48 = jnp.zeros((ZR, 48), jnp.float32)

    idx2d = jnp.concatenate([src2d_d, dst2d_d], axis=0)
    cnt = _make_deg()(idx2d, zflat).reshape(NC, NS, NP // D, D)
    grid_s, grid_d = _tc_call(_degsum_body, [(NP // D, D), (NP // D, D)])(
        cnt[0], cnt[1]
    )
    deg_s = grid_s.reshape(-1)[:N].reshape(N, 1)
    deg_d = grid_d.reshape(-1)[:N].reshape(N, 1)

    inv_s, inv_d, xs = _tc_rowblock(
        _prep_body, [(N, 1), (N, 1), (N, D)], [(N, 1), (N, 1), (N, D)]
    )(deg_s, deg_d, x)

    p0, p1 = _make_agg_esplit()(xs, src2d, dst2d, z128)

    g0, g1 = _tc_rowblock(
        _mix1_body,
        [(N, D), (N, D), (N, 1), (N, 1), (D, R * H)],
        [(N, D), (N, D)],
    )(p0, p1, inv_d, inv_s, w1cat)

    y20, y21 = _make_agg_fsplit()(g0, g1, src2d, dst2d, z128)

    qp = _tc_rowblock(
        _mix2_body,
        [(N, D), (N, D), (N, 1), (N, 1), (R * H, R * H), (R * H, 48)],
        [(N, 48)],
    )(y20, y21, inv_d, inv_s, w2bd, wcat)[0]

    y3a, y3b = _make_agg_esplit(48, False)(qp, src2d, dst2d, z48)

    z_mean, z_var = _tc_rowblock(
        _fin_body,
        [(N, 48), (N, 48), (N, 1)],
        [(N, LD), (N, 1)],
    )(y3a, y3b, inv_d)

    return z_mean, z_mean, z_var


# speedup vs baseline: 27.0911x; 1.0139x over previous
import functools

import jax
import jax.numpy as jnp
from jax import lax
from jax.experimental import pallas as pl
from jax.experimental.pallas import tpu as pltpu
from jax.experimental.pallas import tpu_sc as plsc

N = 10000
E = 320000
D = 128
R = 4
H = 64
LD = 32

NC = 2
NS = 16
ECA = 100
RA = E // ECA
CPTA = 208
CPTA2 = 104
RPA = NS * CPTA
GA = 8
ECD = 128
RD = E // ECD
CPTD = 160
RPD = NS * CPTD
ZR = 200
NZC = N // ZR
NP = 10240


def _sc_mesh():
    return plsc.VectorSubcoreMesh(
        core_axis_name="c", subcore_axis_name="s", num_cores=NC, num_subcores=NS
    )


def _zero_acc(s, zrows, acc):
    for j in range(4):
        idx = s + NS * j

        @pl.when(idx < NZC)
        def _():
            pltpu.sync_copy(zrows, acc.at[pl.ds(idx * ZR, ZR)])


def _write_out(s, acc, outh):
    for j in range(4):
        idx = s + NS * j

        @pl.when(idx < NZC)
        def _():
            pltpu.sync_copy(acc.at[pl.ds(idx * ZR, ZR)], outh.at[pl.ds(idx * ZR, ZR)])


def _agg_loop(nck, tilebase, xh, src2d, dst2d, srcb, dstb, rows, acc,
              isem, gsem, ssem):
    def idx_start(g, p):
        base = tilebase + g * GA
        pltpu.async_copy(src2d.at[pl.ds(base, GA)], srcb.at[p], isem.at[p])
        pltpu.async_copy(dst2d.at[pl.ds(base, GA)], dstb.at[p], isem.at[p])

    def idx_wait(g, p):
        base = tilebase + g * GA
        pltpu.make_async_copy(src2d.at[pl.ds(base, GA)], srcb.at[p], isem.at[p]).wait()
        pltpu.make_async_copy(dst2d.at[pl.ds(base, GA)], dstb.at[p], isem.at[p]).wait()

    def g_start(m):
        p = jnp.bitwise_and(lax.shift_right_logical(m, 3), 1)
        j = jnp.bitwise_and(m, GA - 1)
        b = lax.rem(m, 3)
        pltpu.async_copy(xh.at[srcb.at[p, j]], rows.at[b], gsem.at[b])

    def g_wait(m):
        p = jnp.bitwise_and(lax.shift_right_logical(m, 3), 1)
        j = jnp.bitwise_and(m, GA - 1)
        b = lax.rem(m, 3)
        pltpu.make_async_copy(xh.at[srcb.at[p, j]], rows.at[b], gsem.at[b]).wait()

    def s_start(m):
        p = jnp.bitwise_and(lax.shift_right_logical(m, 3), 1)
        j = jnp.bitwise_and(m, GA - 1)
        b = lax.rem(m, 3)
        pltpu.async_copy(rows.at[b], acc.at[dstb.at[p, j]], ssem.at[b], add=True)

    def s_wait(m):
        p = jnp.bitwise_and(lax.shift_right_logical(m, 3), 1)
        j = jnp.bitwise_and(m, GA - 1)
        b = lax.rem(m, 3)
        pltpu.make_async_copy(rows.at[b], acc.at[dstb.at[p, j]], ssem.at[b]).wait()

    idx_start(0, 0)
    idx_wait(0, 0)
    pl.when(nck > 0)(lambda: g_start(0))

    def body(k, carry):
        j = jnp.bitwise_and(k, GA - 1)
        g = lax.shift_right_logical(k, 3)
        p = jnp.bitwise_and(g, 1)

        pl.when(jnp.logical_and(j == 2, (g + 1) * GA < nck))(
            lambda: idx_start(g + 1, 1 - p)
        )
        pl.when(jnp.logical_and(j == GA - 1, k + 1 < nck))(
            lambda: idx_wait(g + 1, 1 - p)
        )

        @pl.when(k + 1 < nck)
        def _():
            pl.when(k >= 2)(lambda: s_wait(k - 2))
            g_start(k + 1)

        g_wait(k)
        s_start(k)
        return carry

    lax.fori_loop(0, nck, body, None)
    for d in (3, 2, 1):
        pl.when(nck >= d)(lambda d=d: s_wait(nck - d))


def _agg_scratch(fw):
    return [
        pltpu.VMEM((2, GA, ECA), jnp.int32),
        pltpu.VMEM((2, GA, ECA), jnp.int32),
        pltpu.VMEM((3, ECA, fw), jnp.float32),
        pltpu.VMEM_SHARED((N, fw), jnp.float32),
        pltpu.SemaphoreType.DMA((2,)),
        pltpu.SemaphoreType.DMA((3,)),
        pltpu.SemaphoreType.DMA((3,)),
    ]


def _make_agg_esplit(fw=D, tc_tiling=True):
    @functools.partial(
        pl.kernel,
        out_type=(
            jax.ShapeDtypeStruct((N, fw), jnp.float32),
            jax.ShapeDtypeStruct((N, fw), jnp.float32),
        ),
        mesh=_sc_mesh(),
        scratch_types=_agg_scratch(fw),
        compiler_params=pltpu.CompilerParams(use_tc_tiling_on_sc=tc_tiling),
    )
    def agg(x, src2d, dst2d, zrows, out0, out1, srcb, dstb, rows, acc, isem, gsem, ssem):
        c = lax.axis_index("c")
        s = lax.axis_index("s")
        w = c * NS + s

        _zero_acc(s, zrows, acc)
        nck = jnp.clip(RA - w * CPTA2, 0, CPTA2)

        plsc.subcore_barrier()
        _agg_loop(nck, w * CPTA2, x, src2d, dst2d, srcb, dstb, rows, acc,
                  isem, gsem, ssem)
        plsc.subcore_barrier()

        pl.when(c == 0)(lambda: _write_out(s, acc, out0))
        pl.when(c == 1)(lambda: _write_out(s, acc, out1))

    return agg


def _make_agg_fsplit():
    @functools.partial(
        pl.kernel,
        out_type=(
            jax.ShapeDtypeStruct((N, D), jnp.float32),
            jax.ShapeDtypeStruct((N, D), jnp.float32),
        ),
        mesh=_sc_mesh(),
        scratch_types=_agg_scratch(D),
    )
    def agg(x0, x1, src2d, dst2d, zrows, out0, out1, srcb, dstb, rows, acc,
            isem, gsem, ssem):
        c = lax.axis_index("c")
        s = lax.axis_index("s")

        _zero_acc(s, zrows, acc)
        nck = jnp.clip(RA - s * CPTA, 0, CPTA)

        plsc.subcore_barrier()

        def run(xh, outh):
            _agg_loop(nck, s * CPTA, xh, src2d, dst2d, srcb, dstb, rows, acc,
                      isem, gsem, ssem)
            plsc.subcore_barrier()
            _write_out(s, acc, outh)

        pl.when(c == 0)(lambda: run(x0, out0))
        pl.when(c == 1)(lambda: run(x1, out1))

    return agg


def _make_deg():
    @functools.partial(
        pl.kernel,
        out_type=jax.ShapeDtypeStruct((NC * NS * NP,), jnp.float32),
        mesh=_sc_mesh(),
        scratch_types=[
            pltpu.VMEM((CPTD, ECD), jnp.int32),
            pltpu.VMEM((NP,), jnp.float32),
            pltpu.SemaphoreType.DMA((2,)),
        ],
        compiler_params=pltpu.CompilerParams(needs_layout_passes=False),
    )
    def deg(idx2d, zflat, out, idxb, counts, dsem):
        c = lax.axis_index("c")
        s = lax.axis_index("s")
        w = c * NS + s

        pltpu.async_copy(zflat, counts, dsem.at[0]).wait()
        pltpu.async_copy(
            idx2d.at[pl.ds(c * RPD + s * CPTD, CPTD)], idxb, dsem.at[1]
        ).wait()
        nck = jnp.clip(RD - s * CPTD, 0, CPTD)

        ones16 = jnp.full((16,), 1.0, jnp.float32)

        def body(k, carry):
            for j in range(ECD // 16):
                v = idxb[k, pl.ds(j * 16, 16)]
                plsc.addupdate_scatter(counts, [v], ones16)
            return carry

        lax.fori_loop(0, nck, body, None)

        pltpu.async_copy(counts, out.at[pl.ds(w * NP, NP)], dsem.at[0]).wait()

    return deg


def _dot(a, b):
    return jnp.dot(a, b, preferred_element_type=jnp.float32)


def _tc_call(body, out_shapes):
    return pl.pallas_call(
        body,
        out_shape=tuple(jax.ShapeDtypeStruct(s, jnp.float32) for s in out_shapes),
    )


BR = 2000


def _rb_spec(shape):
    if shape[0] == N:
        nd = len(shape)
        return pl.BlockSpec((BR,) + shape[1:], lambda i: (i,) + (0,) * (nd - 1))
    return pl.BlockSpec(shape, lambda i: (0,) * len(shape))


def _tc_rowblock(body, in_shapes, out_shapes):
    return pl.pallas_call(
        body,
        grid=(N // BR,),
        in_specs=[_rb_spec(s) for s in in_shapes],
        out_specs=tuple(_rb_spec(s) for s in out_shapes),
        out_shape=tuple(jax.ShapeDtypeStruct(s, jnp.float32) for s in out_shapes),
    )


def _degsum_body(csr, cdr, gs_r, gd_r):
    gs_r[...] = jnp.sum(csr[...], axis=0)
    gd_r[...] = jnp.sum(cdr[...], axis=0)


def _prep_body(dsr, ddr, xr, inv_s_r, inv_d_r, xs_r):
    ds = dsr[...]
    dd = ddr[...]
    inv_s = jnp.where(ds > 0, lax.rsqrt(jnp.maximum(ds, 1.0)), 0.0)
    inv_d = jnp.where(dd > 0, lax.rsqrt(jnp.maximum(dd, 1.0)), 0.0)
    inv_s_r[...] = inv_s
    inv_d_r[...] = inv_d
    xs_r[...] = xr[...] * inv_s


def _mix1_body(p0r, p1r, invdr, invsr, wr, g0r, g1r):
    y = (p0r[...] + p1r[...]) * invdr[...]
    g = jnp.tanh(_dot(y, wr[...])) * invsr[...]
    g0r[...] = g[:, :D]
    g1r[...] = g[:, D:]


def _mix2_body(y0r, y1r, invdr, invsr, w2r, wcr, qr):
    y = jnp.concatenate([y0r[...], y1r[...]], axis=1) * invdr[...]
    z = jnp.tanh(_dot(y, w2r[...])) * invsr[...]
    qr[...] = _dot(z, wcr[...])


def _fin_body(y0r, y1r, invdr, zmr, zvr):
    yw = (y0r[...] + y1r[...]) * invdr[...]
    p = yw[:, :LD]
    nrm = jnp.sqrt(jnp.sum(p * p, axis=1, keepdims=True))
    zmr[...] = p / (1e-4 + nrm)
    v = yw[:, LD:LD + 1]
    zvr[...] = jnp.log1p(jnp.exp(-jnp.abs(v))) + jnp.maximum(v, 0.0) + 1.0


@jax.jit
def kernel(x, edge_index, Ws1, Ws2, Wm, Ws):
    ei = edge_index.astype(jnp.int32)
    eia = jnp.concatenate([ei, jnp.zeros((2, RPA * ECA - E), jnp.int32)], axis=1)
    src2d = eia[0].reshape(RPA, ECA)
    dst2d = eia[1].reshape(RPA, ECA)
    eid = jnp.concatenate([ei, jnp.zeros((2, RPD * ECD - E), jnp.int32)], axis=1)
    src2d_d = eid[0].reshape(RPD, ECD)
    dst2d_d = eid[1].reshape(RPD, ECD)

    w1cat = jnp.concatenate(Ws1, axis=1)
    w2bd = jnp.zeros((R * H, R * H), jnp.float32)
    for r in range(R):
        w2bd = w2bd.at[r * H:(r + 1) * H, r * H:(r + 1) * H].set(Ws2[r])
    wcat = jnp.zeros((R * H, 48), jnp.float32)
    wcat = wcat.at[:, :LD].set(Wm).at[:, LD].set(Ws[:, 0])

    zflat = jnp.zeros((NP,), jnp.float32)
    z128 = jnp.zeros((ZR, D), jnp.float32)
    z48 = jnp.zeros((ZR, 48), jnp.float32)

    idx2d = jnp.concatenate([src2d_d, dst2d_d], axis=0)
    cnt = _make_deg()(idx2d, zflat).reshape(NC, NS, NP // D, D)
    grid_s, grid_d = _tc_call(_degsum_body, [(NP // D, D), (NP // D, D)])(
        cnt[0], cnt[1]
    )
    deg_s = grid_s.reshape(-1)[:N].reshape(N, 1)
    deg_d = grid_d.reshape(-1)[:N].reshape(N, 1)

    inv_s, inv_d, xs = _tc_rowblock(
        _prep_body, [(N, 1), (N, 1), (N, D)], [(N, 1), (N, 1), (N, D)]
    )(deg_s, deg_d, x)

    p0, p1 = _make_agg_esplit()(xs, src2d, dst2d, z128)

    g0, g1 = _tc_rowblock(
        _mix1_body,
        [(N, D), (N, D), (N, 1), (N, 1), (D, R * H)],
        [(N, D), (N, D)],
    )(p0, p1, inv_d, inv_s, w1cat)

    y20, y21 = _make_agg_fsplit()(g0, g1, src2d, dst2d, z128)

    qp = _tc_rowblock(
        _mix2_body,
        [(N, D), (N, D), (N, 1), (N, 1), (R * H, R * H), (R * H, 48)],
        [(N, 48)],
    )(y20, y21, inv_d, inv_s, w2bd, wcat)[0]

    y3a, y3b = _make_agg_esplit(48, False)(qp, src2d, dst2d, z48)

    z_mean, z_var = _tc_rowblock(
        _fin_body,
        [(N, 48), (N, 48), (N, 1)],
        [(N, LD), (N, 1)],
    )(y3a, y3b, inv_d)

    return z_mean, z_mean, z_var


# speedup vs baseline: 27.4608x; 1.0136x over previous
import functools

import jax
import jax.numpy as jnp
from jax import lax
from jax.experimental import pallas as pl
from jax.experimental.pallas import tpu as pltpu
from jax.experimental.pallas import tpu_sc as plsc

N = 10000
E = 320000
D = 128
R = 4
H = 64
LD = 32

NC = 2
NS = 16
ECA = 100
RA = E // ECA
CPTA = 200
CPTA2 = 104
RPA = NS * CPTA
GA = 8
ECD = 128
RD = E // ECD
CPTD = 160
RPD = NS * CPTD
ZR = 200
NZC = N // ZR
NP = 10240


def _sc_mesh():
    return plsc.VectorSubcoreMesh(
        core_axis_name="c", subcore_axis_name="s", num_cores=NC, num_subcores=NS
    )


def _zero_acc(s, zrows, acc):
    for j in range(4):
        idx = s + NS * j

        @pl.when(idx < NZC)
        def _():
            pltpu.sync_copy(zrows, acc.at[pl.ds(idx * ZR, ZR)])


def _write_out(s, acc, outh):
    for j in range(4):
        idx = s + NS * j

        @pl.when(idx < NZC)
        def _():
            pltpu.sync_copy(acc.at[pl.ds(idx * ZR, ZR)], outh.at[pl.ds(idx * ZR, ZR)])


def _agg_loop(nck, tilebase, xh, src2d, dst2d, srcb, dstb, rows, acc,
              isem, gsem, ssem):
    def idx_start(g, p):
        base = tilebase + g * GA
        pltpu.async_copy(src2d.at[pl.ds(base, GA)], srcb.at[p], isem.at[p])
        pltpu.async_copy(dst2d.at[pl.ds(base, GA)], dstb.at[p], isem.at[p])

    def idx_wait(g, p):
        base = tilebase + g * GA
        pltpu.make_async_copy(src2d.at[pl.ds(base, GA)], srcb.at[p], isem.at[p]).wait()
        pltpu.make_async_copy(dst2d.at[pl.ds(base, GA)], dstb.at[p], isem.at[p]).wait()

    def g_start(m):
        p = jnp.bitwise_and(lax.shift_right_logical(m, 3), 1)
        j = jnp.bitwise_and(m, GA - 1)
        b = lax.rem(m, 3)
        pltpu.async_copy(xh.at[srcb.at[p, j]], rows.at[b], gsem.at[b])

    def g_wait(m):
        p = jnp.bitwise_and(lax.shift_right_logical(m, 3), 1)
        j = jnp.bitwise_and(m, GA - 1)
        b = lax.rem(m, 3)
        pltpu.make_async_copy(xh.at[srcb.at[p, j]], rows.at[b], gsem.at[b]).wait()

    def s_start(m):
        p = jnp.bitwise_and(lax.shift_right_logical(m, 3), 1)
        j = jnp.bitwise_and(m, GA - 1)
        b = lax.rem(m, 3)
        pltpu.async_copy(rows.at[b], acc.at[dstb.at[p, j]], ssem.at[b], add=True)

    def s_wait(m):
        p = jnp.bitwise_and(lax.shift_right_logical(m, 3), 1)
        j = jnp.bitwise_and(m, GA - 1)
        b = lax.rem(m, 3)
        pltpu.make_async_copy(rows.at[b], acc.at[dstb.at[p, j]], ssem.at[b]).wait()

    idx_start(0, 0)
    idx_wait(0, 0)
    pl.when(nck > 0)(lambda: g_start(0))

    def body(k, carry):
        j = jnp.bitwise_and(k, GA - 1)
        g = lax.shift_right_logical(k, 3)
        p = jnp.bitwise_and(g, 1)

        pl.when(jnp.logical_and(j == 2, (g + 1) * GA < nck))(
            lambda: idx_start(g + 1, 1 - p)
        )
        pl.when(jnp.logical_and(j == GA - 1, k + 1 < nck))(
            lambda: idx_wait(g + 1, 1 - p)
        )

        @pl.when(k + 1 < nck)
        def _():
            pl.when(k >= 2)(lambda: s_wait(k - 2))
            g_start(k + 1)

        g_wait(k)
        s_start(k)
        return carry

    lax.fori_loop(0, nck, body, None)
    for d in (3, 2, 1):
        pl.when(nck >= d)(lambda d=d: s_wait(nck - d))


def _agg_scratch(fw):
    return [
        pltpu.VMEM((2, GA, ECA), jnp.int32),
        pltpu.VMEM((2, GA, ECA), jnp.int32),
        pltpu.VMEM((3, ECA, fw), jnp.float32),
        pltpu.VMEM_SHARED((N, fw), jnp.float32),
        pltpu.SemaphoreType.DMA((2,)),
        pltpu.SemaphoreType.DMA((3,)),
        pltpu.SemaphoreType.DMA((3,)),
    ]


def _make_agg_esplit(fw=D, tc_tiling=True):
    @functools.partial(
        pl.kernel,
        out_type=(
            jax.ShapeDtypeStruct((N, fw), jnp.float32),
            jax.ShapeDtypeStruct((N, fw), jnp.float32),
        ),
        mesh=_sc_mesh(),
        scratch_types=_agg_scratch(fw),
        compiler_params=pltpu.CompilerParams(use_tc_tiling_on_sc=tc_tiling),
    )
    def agg(x, src2d, dst2d, zrows, out0, out1, srcb, dstb, rows, acc, isem, gsem, ssem):
        c = lax.axis_index("c")
        s = lax.axis_index("s")
        w = c * NS + s

        _zero_acc(s, zrows, acc)
        nck = jnp.clip(RA - w * CPTA2, 0, CPTA2)

        plsc.subcore_barrier()
        _agg_loop(nck, w * CPTA2, x, src2d, dst2d, srcb, dstb, rows, acc,
                  isem, gsem, ssem)
        plsc.subcore_barrier()

        pl.when(c == 0)(lambda: _write_out(s, acc, out0))
        pl.when(c == 1)(lambda: _write_out(s, acc, out1))

    return agg


def _make_agg_fsplit():
    @functools.partial(
        pl.kernel,
        out_type=(
            jax.ShapeDtypeStruct((N, D), jnp.float32),
            jax.ShapeDtypeStruct((N, D), jnp.float32),
        ),
        mesh=_sc_mesh(),
        scratch_types=_agg_scratch(D),
    )
    def agg(x0, x1, src2d, dst2d, zrows, out0, out1, srcb, dstb, rows, acc,
            isem, gsem, ssem):
        c = lax.axis_index("c")
        s = lax.axis_index("s")

        _zero_acc(s, zrows, acc)
        nck = jnp.clip(RA - s * CPTA, 0, CPTA)

        plsc.subcore_barrier()

        def run(xh, outh):
            _agg_loop(nck, s * CPTA, xh, src2d, dst2d, srcb, dstb, rows, acc,
                      isem, gsem, ssem)
            plsc.subcore_barrier()
            _write_out(s, acc, outh)

        pl.when(c == 0)(lambda: run(x0, out0))
        pl.when(c == 1)(lambda: run(x1, out1))

    return agg


def _make_deg():
    @functools.partial(
        pl.kernel,
        out_type=jax.ShapeDtypeStruct((NC * NS * NP,), jnp.float32),
        mesh=_sc_mesh(),
        scratch_types=[
            pltpu.VMEM((CPTD, ECD), jnp.int32),
            pltpu.VMEM((NP,), jnp.float32),
            pltpu.SemaphoreType.DMA((2,)),
        ],
        compiler_params=pltpu.CompilerParams(needs_layout_passes=False),
    )
    def deg(idx2d, zflat, out, idxb, counts, dsem):
        c = lax.axis_index("c")
        s = lax.axis_index("s")
        w = c * NS + s

        pltpu.async_copy(zflat, counts, dsem.at[0]).wait()
        pltpu.async_copy(
            idx2d.at[pl.ds(c * RPD + s * CPTD, CPTD)], idxb, dsem.at[1]
        ).wait()
        nck = jnp.clip(RD - s * CPTD, 0, CPTD)

        ones16 = jnp.full((16,), 1.0, jnp.float32)

        def body(k, carry):
            for j in range(ECD // 16):
                v = idxb[k, pl.ds(j * 16, 16)]
                plsc.addupdate_scatter(counts, [v], ones16)
            return carry

        lax.fori_loop(0, nck, body, None)

        pltpu.async_copy(counts, out.at[pl.ds(w * NP, NP)], dsem.at[0]).wait()

    return deg


def _dot(a, b):
    return jnp.dot(a, b, preferred_element_type=jnp.float32)


def _tc_call(body, out_shapes):
    return pl.pallas_call(
        body,
        out_shape=tuple(jax.ShapeDtypeStruct(s, jnp.float32) for s in out_shapes),
    )


BR = 2000


def _rb_spec(shape):
    if shape[0] == N:
        nd = len(shape)
        return pl.BlockSpec((BR,) + shape[1:], lambda i: (i,) + (0,) * (nd - 1))
    return pl.BlockSpec(shape, lambda i: (0,) * len(shape))


def _tc_rowblock(body, in_shapes, out_shapes):
    return pl.pallas_call(
        body,
        grid=(N // BR,),
        in_specs=[_rb_spec(s) for s in in_shapes],
        out_specs=tuple(_rb_spec(s) for s in out_shapes),
        out_shape=tuple(jax.ShapeDtypeStruct(s, jnp.float32) for s in out_shapes),
    )


def _degsum_body(csr, cdr, gs_r, gd_r):
    gs_r[...] = jnp.sum(csr[...], axis=0)
    gd_r[...] = jnp.sum(cdr[...], axis=0)


def _prep_body(dsr, ddr, xr, inv_s_r, inv_d_r, xs_r):
    ds = dsr[...]
    dd = ddr[...]
    inv_s = jnp.where(ds > 0, lax.rsqrt(jnp.maximum(ds, 1.0)), 0.0)
    inv_d = jnp.where(dd > 0, lax.rsqrt(jnp.maximum(dd, 1.0)), 0.0)
    inv_s_r[...] = inv_s
    inv_d_r[...] = inv_d
    xs_r[...] = xr[...] * inv_s


def _mix1_body(p0r, p1r, invdr, invsr, wr, g0r, g1r):
    y = (p0r[...] + p1r[...]) * invdr[...]
    g = jnp.tanh(_dot(y, wr[...])) * invsr[...]
    g0r[...] = g[:, :D]
    g1r[...] = g[:, D:]


def _mix2_body(y0r, y1r, invdr, invsr, w2r, wcr, qr):
    y = jnp.concatenate([y0r[...], y1r[...]], axis=1) * invdr[...]
    z = jnp.tanh(_dot(y, w2r[...])) * invsr[...]
    qr[...] = _dot(z, wcr[...])


def _fin_body(y0r, y1r, invdr, zmr, zvr):
    yw = (y0r[...] + y1r[...]) * invdr[...]
    p = yw[:, :LD]
    nrm = jnp.sqrt(jnp.sum(p * p, axis=1, keepdims=True))
    zmr[...] = p / (1e-4 + nrm)
    v = yw[:, LD:LD + 1]
    zvr[...] = jnp.log1p(jnp.exp(-jnp.abs(v))) + jnp.maximum(v, 0.0) + 1.0


@jax.jit
def kernel(x, edge_index, Ws1, Ws2, Wm, Ws):
    ei = edge_index.astype(jnp.int32)
    eia = jnp.concatenate([ei, jnp.zeros((2, RPA * ECA - E), jnp.int32)], axis=1)
    src2d = eia[0].reshape(RPA, ECA)
    dst2d = eia[1].reshape(RPA, ECA)
    eid = jnp.concatenate([ei, jnp.zeros((2, RPD * ECD - E), jnp.int32)], axis=1)
    src2d_d = eid[0].reshape(RPD, ECD)
    dst2d_d = eid[1].reshape(RPD, ECD)

    w1cat = jnp.concatenate(Ws1, axis=1)
    w2bd = jnp.zeros((R * H, R * H), jnp.float32)
    for r in range(R):
        w2bd = w2bd.at[r * H:(r + 1) * H, r * H:(r + 1) * H].set(Ws2[r])
    wcat = jnp.zeros((R * H, 48), jnp.float32)
    wcat = wcat.at[:, :LD].set(Wm).at[:, LD].set(Ws[:, 0])

    zflat = jnp.zeros((NP,), jnp.float32)
    z128 = jnp.zeros((ZR, D), jnp.float32)
    z48 = jnp.zeros((ZR, 48), jnp.float32)

    idx2d = jnp.concatenate([src2d_d, dst2d_d], axis=0)
    cnt = _make_deg()(idx2d, zflat).reshape(NC, NS, NP // D, D)
    grid_s, grid_d = _tc_call(_degsum_body, [(NP // D, D), (NP // D, D)])(
        cnt[0], cnt[1]
    )
    deg_s = grid_s.reshape(-1)[:N].reshape(N, 1)
    deg_d = grid_d.reshape(-1)[:N].reshape(N, 1)

    inv_s, inv_d, xs = _tc_rowblock(
        _prep_body, [(N, 1), (N, 1), (N, D)], [(N, 1), (N, 1), (N, D)]
    )(deg_s, deg_d, x)

    p0, p1 = _make_agg_esplit()(xs, src2d, dst2d, z128)

    g0, g1 = _tc_rowblock(
        _mix1_body,
        [(N, D), (N, D), (N, 1), (N, 1), (D, R * H)],
        [(N, D), (N, D)],
    )(p0, p1, inv_d, inv_s, w1cat)

    y20, y21 = _make_agg_fsplit()(g0, g1, src2d, dst2d, z128)

    qp = _tc_rowblock(
        _mix2_body,
        [(N, D), (N, D), (N, 1), (N, 1), (R * H, R * H), (R * H, 48)],
        [(N, 48)],
    )(y20, y21, inv_d, inv_s, w2bd, wcat)[0]

    y3a, y3b = _make_agg_esplit(48, False)(qp, src2d, dst2d, z48)

    z_mean, z_var = _tc_rowblock(
        _fin_body,
        [(N, 48), (N, 48), (N, 1)],
        [(N, LD), (N, 1)],
    )(y3a, y3b, inv_d)

    return z_mean, z_mean, z_var
